# baseline (speedup 1.0000x reference)
import jax
import jax.numpy as jnp
from jax import lax
from jax.experimental import pallas as pl
from jax.experimental.pallas import tpu as pltpu
from jax.experimental.pallas import tpu_sc as plsc

C = 160
NPTS = 500
NP_NEIGH = 8
L = 16
NPAD = 512
NC = 2
NS = 16
NW = NC * NS
CPW = C // NW
NBLK = NPAD // L


def _sc_body(f1_hbm, f2_hbm, inds_hbm, out_hbm,
             f1_v, f2_v, inds_v, out_v, sem1, sem2, sem3):
    wid = lax.axis_index("s") * NC + lax.axis_index("c")
    e0 = wid * (CPW * NPAD)
    cp1 = pltpu.async_copy(f2_hbm.at[pl.ds(e0, CPW * NPAD)], f2_v, sem1)
    cp2 = pltpu.async_copy(inds_hbm, inds_v, sem2)
    cp3 = pltpu.async_copy(f1_hbm.at[pl.ds(e0, CPW * NPAD)], f1_v, sem3)
    cp1.wait()
    cp2.wait()

    cp3.wait()

    @plsc.parallel_loop(0, NBLK, unroll=4)
    def block(blk):
        base = blk * L
        idxs = [inds_v[pl.ds(n * NPAD + base, L)] for n in range(NP_NEIGH)]
        for c in range(CPW):
            roff = c * NPAD
            g = [plsc.load_gather(f2_v, [idxs[n] + roff])
                 for n in range(NP_NEIGH)]
            s0 = (g[0] + g[1]) + (g[2] + g[3])
            s1 = (g[4] + g[5]) + (g[6] + g[7])
            ds = pl.ds(roff + base, L)
            out_v[ds] = (s0 + s1) * f1_v[ds] * 0.125
    pltpu.sync_copy(out_v, out_hbm.at[pl.ds(e0, CPW * NPAD)])


@jax.jit
def kernel(feat1, feat2, inds):
    f1 = jnp.pad(feat1[0], ((0, 0), (0, NPAD - NPTS))).reshape(-1)
    f2 = jnp.pad(feat2[0], ((0, 0), (0, NPAD - NPTS))).reshape(-1)
    it = jnp.pad(inds[0].astype(jnp.int32).T,
                 ((0, 0), (0, NPAD - NPTS))).reshape(-1)

    mesh = plsc.VectorSubcoreMesh(core_axis_name="c", subcore_axis_name="s")
    out = pl.kernel(
        _sc_body,
        out_type=jax.ShapeDtypeStruct((C * NPAD,), jnp.float32),
        mesh=mesh,
        compiler_params=pltpu.CompilerParams(needs_layout_passes=False),
        scratch_types=[
            pltpu.VMEM((CPW * NPAD,), jnp.float32),
            pltpu.VMEM((CPW * NPAD,), jnp.float32),
            pltpu.VMEM((NP_NEIGH * NPAD,), jnp.int32),
            pltpu.VMEM((CPW * NPAD,), jnp.float32),
            pltpu.SemaphoreType.DMA,
            pltpu.SemaphoreType.DMA,
            pltpu.SemaphoreType.DMA,
        ],
    )(f1, f2, it)
    return out.reshape(1, C, NPAD)[:, :, :NPTS]


# speedup vs baseline: 1.0513x; 1.0513x over previous
import jax
import jax.numpy as jnp
from jax import lax
from jax.experimental import pallas as pl
from jax.experimental.pallas import tpu as pltpu
from jax.experimental.pallas import tpu_sc as plsc

C = 160
NPTS = 500
NP_NEIGH = 8
L = 16
NPAD = 512
NC = 2
NS = 16
NW = NC * NS
CPW = C // NW
NBLK = NPAD // L


def _sc_body(f1_hbm, f2_hbm, inds_hbm, out_hbm,
             f1_v, f2_v, inds_v, out_v, sem1, sem2, sem3):
    wid = lax.axis_index("s") * NC + lax.axis_index("c")
    e0 = wid * (CPW * NPAD)
    cp1 = pltpu.async_copy(f2_hbm.at[pl.ds(e0, CPW * NPAD)], f2_v, sem1)
    cp2 = pltpu.async_copy(inds_hbm, inds_v, sem2)
    cp3 = pltpu.async_copy(f1_hbm.at[pl.ds(e0, CPW * NPAD)], f1_v, sem3)
    cp1.wait()
    cp2.wait()

    cp3.wait()

    @plsc.parallel_loop(0, NBLK, unroll=1)
    def block(blk):
        base = blk * L
        idxs = [inds_v[pl.ds(n * NPAD + base, L)] for n in range(NP_NEIGH)]
        for c in range(CPW):
            roff = c * NPAD
            g = [plsc.load_gather(f2_v, [idxs[n] + roff])
                 for n in range(NP_NEIGH)]
            s0 = (g[0] + g[1]) + (g[2] + g[3])
            s1 = (g[4] + g[5]) + (g[6] + g[7])
            ds = pl.ds(roff + base, L)
            out_v[ds] = (s0 + s1) * f1_v[ds] * 0.125
    pltpu.sync_copy(out_v, out_hbm.at[pl.ds(e0, CPW * NPAD)])


@jax.jit
def kernel(feat1, feat2, inds):
    f1 = jnp.pad(feat1[0], ((0, 0), (0, NPAD - NPTS))).reshape(-1)
    f2 = jnp.pad(feat2[0], ((0, 0), (0, NPAD - NPTS))).reshape(-1)
    it = jnp.pad(inds[0].astype(jnp.int32).T,
                 ((0, 0), (0, NPAD - NPTS))).reshape(-1)

    mesh = plsc.VectorSubcoreMesh(core_axis_name="c", subcore_axis_name="s")
    out = pl.kernel(
        _sc_body,
        out_type=jax.ShapeDtypeStruct((C * NPAD,), jnp.float32),
        mesh=mesh,
        compiler_params=pltpu.CompilerParams(needs_layout_passes=False),
        scratch_types=[
            pltpu.VMEM((CPW * NPAD,), jnp.float32),
            pltpu.VMEM((CPW * NPAD,), jnp.float32),
            pltpu.VMEM((NP_NEIGH * NPAD,), jnp.int32),
            pltpu.VMEM((CPW * NPAD,), jnp.float32),
            pltpu.SemaphoreType.DMA,
            pltpu.SemaphoreType.DMA,
            pltpu.SemaphoreType.DMA,
        ],
    )(f1, f2, it)
    return out.reshape(1, C, NPAD)[:, :, :NPTS]
